# baseline (speedup 1.0000x reference)
import functools
import jax
import jax.numpy as jnp
from jax import lax
from jax.experimental import pallas as pl
from jax.experimental.pallas import tpu as pltpu
from jax.experimental.pallas import tpu_sc as plsc
from jax.experimental.layout import Format, Layout

VOCAB = 1000000
D = 64
SCALE = 8.0

NC = 2
NS = 16
NW = NC * NS

B_TOTAL = 4096 * 50
B_PER_W = B_TOTAL // NW
C = 128
NCHUNK = B_PER_W // C
NBUF = 5
N_OUTER = NCHUNK // NBUF


def _make_kernel():
    mesh = plsc.VectorSubcoreMesh(core_axis_name="c", subcore_axis_name="s")

    @functools.partial(
        pl.kernel,
        mesh=mesh,
        out_type=jax.ShapeDtypeStruct((B_TOTAL, D), jnp.float32),
        scratch_types=[
            pltpu.VMEM((NCHUNK, C), jnp.int32),
            pltpu.VMEM((NBUF, C, D), jnp.float32),
            pltpu.VMEM((NBUF, C, D), jnp.float32),
        ]
        + [pltpu.SemaphoreType.DMA] * (2 * NBUF),
        compiler_params=pltpu.CompilerParams(use_tc_tiling_on_sc=False),
    )
    def emb_kernel(idx_hbm, lut_hbm, out_hbm, idx_v, gbuf, wbuf, *sems):
        gsems = sems[:NBUF]
        wsems = sems[NBUF:]
        wid = lax.axis_index("s") * NC + lax.axis_index("c")
        base = wid * B_PER_W
        pltpu.sync_copy(idx_hbm.at[wid], idx_v)

        def gather(c, b):
            return pltpu.async_copy(
                lut_hbm.at[idx_v.at[c]], gbuf.at[b], gsems[b])

        for b in range(NBUF):
            gather(b, b)

        def outer(g, carry):
            for b in range(NBUF):
                c = g * NBUF + b
                pltpu.make_async_copy(
                    lut_hbm.at[idx_v.at[c]], gbuf.at[b], gsems[b]).wait()
                @pl.when(g > 0)
                def _():
                    pltpu.make_async_copy(
                        wbuf.at[b],
                        out_hbm.at[pl.ds(base, C)],
                        wsems[b]).wait()

                def row_body(i, carry2):
                    for j in range(D // 16):
                        sl = pl.ds(j * 16, 16)
                        wbuf[b, i, sl] = gbuf[b, i, sl] * SCALE
                    return carry2

                lax.fori_loop(0, C, row_body, 0, unroll=4)
                @pl.when(c + NBUF < NCHUNK)
                def _():
                    gather(c + NBUF, b)
                pltpu.async_copy(
                    wbuf.at[b], out_hbm.at[pl.ds(base + c * C, C)], wsems[b])
            return carry

        lax.fori_loop(0, N_OUTER, outer, 0)
        for b in range(NBUF):
            pltpu.make_async_copy(
                wbuf.at[b], out_hbm.at[pl.ds(base, C)], wsems[b]).wait()

    return emb_kernel


_emb = _make_kernel()


def _impl(x, lut):
    idx = x.astype(jnp.int32).reshape(NW, NCHUNK, C)
    out = _emb(idx, lut)
    return out.reshape(x.shape[0], x.shape[1], D)


_jitted = None


def kernel(x, lut):
    global _jitted
    if _jitted is None:
        dev = jax.devices()[0]
        fmt = Format(
            Layout(major_to_minor=(0, 1, 2), tiling=()),
            jax.sharding.SingleDeviceSharding(dev),
        )
        _jitted = jax.jit(_impl, out_shardings=fmt)
    return _jitted(x, lut)


# speedup vs baseline: 1.0231x; 1.0231x over previous
import functools
import jax
import jax.numpy as jnp
from jax import lax
from jax.experimental import pallas as pl
from jax.experimental.pallas import tpu as pltpu
from jax.experimental.pallas import tpu_sc as plsc

VOCAB = 1000000
D = 64
SCALE = 8.0

NC = 2
NS = 16
NW = NC * NS

NROW = 4096
SEQ = 50
R_PER_W = NROW // NW
NBUF = 4
N_OUTER = R_PER_W // NBUF


def _make_kernel():
    mesh = plsc.VectorSubcoreMesh(core_axis_name="c", subcore_axis_name="s")

    @functools.partial(
        pl.kernel,
        mesh=mesh,
        out_type=jax.ShapeDtypeStruct((NROW, SEQ, D), jnp.float32),
        scratch_types=[
            pltpu.VMEM((R_PER_W, SEQ), jnp.int32),
            pltpu.VMEM((NBUF, SEQ, D), jnp.float32),
            pltpu.VMEM((NBUF, SEQ, D), jnp.float32),
        ]
        + [pltpu.SemaphoreType.DMA] * (2 * NBUF),
        compiler_params=pltpu.CompilerParams(use_tc_tiling_on_sc=False),
    )
    def emb_kernel(x_hbm, lut_hbm, out_hbm, xbuf, gbuf, wbuf, *sems):
        gsems = sems[:NBUF]
        wsems = sems[NBUF:]
        wid = lax.axis_index("s") * NC + lax.axis_index("c")
        row0 = wid * R_PER_W
        pltpu.sync_copy(x_hbm.at[pl.ds(row0, R_PER_W)], xbuf)

        def gather(i, b):
            return pltpu.async_copy(
                lut_hbm.at[xbuf.at[i]], gbuf.at[b], gsems[b])

        for b in range(NBUF):
            gather(b, b)

        def outer(g, carry):
            for b in range(NBUF):
                i = g * NBUF + b
                pltpu.make_async_copy(
                    lut_hbm.at[xbuf.at[i]], gbuf.at[b], gsems[b]).wait()
                @pl.when(g > 0)
                def _():
                    pltpu.make_async_copy(
                        wbuf.at[b], out_hbm.at[row0], wsems[b]).wait()
                for r in range(SEQ):
                    for j in range(D // 16):
                        sl = pl.ds(j * 16, 16)
                        wbuf[b, r, sl] = gbuf[b, r, sl] * SCALE
                @pl.when(i + NBUF < R_PER_W)
                def _():
                    gather(i + NBUF, b)
                pltpu.async_copy(
                    wbuf.at[b], out_hbm.at[row0 + i], wsems[b])
            return carry

        lax.fori_loop(0, N_OUTER, outer, 0)
        for b in range(NBUF):
            pltpu.make_async_copy(
                wbuf.at[b], out_hbm.at[row0], wsems[b]).wait()

    return emb_kernel


_emb = _make_kernel()


@jax.jit
def kernel(x, lut):
    return _emb(x.astype(jnp.int32), lut)
